# Initial kernel scaffold; baseline (speedup 1.0000x reference)
import jax
import jax.numpy as jnp
from jax.experimental import pallas as pl


def kernel(indices, table):
    raise NotImplementedError("write your pallas kernel here")


# speedup vs baseline: 1.4980x; 1.4980x over previous
import functools

import jax
import jax.numpy as jnp
from jax import lax
from jax.experimental import pallas as pl
from jax.experimental.pallas import tpu as pltpu
from jax.experimental.pallas import tpu_sc as plsc

VOCAB = 1000000
EMBED_DIM = 32
BATCH = 4096
HIST = 200

_INFO = plsc.get_sparse_core_info()
_NC = _INFO.num_cores
_NS = _INFO.num_subcores
_NW = _NC * _NS
_HB = HIST // 8
_BB = BATCH // 128
_TBYTES = 4 * 8 * 128 * 4


@functools.partial(
    pl.kernel,
    mesh=plsc.VectorSubcoreMesh(core_axis_name="c", subcore_axis_name="s"),
    out_type=jax.ShapeDtypeStruct((HIST, 4, _BB, 8, 128), jnp.float32),
    scratch_types=[
        pltpu.VMEM((_HB, 8, 128), jnp.int32),
        pltpu.VMEM((2, 128, EMBED_DIM), jnp.float32),
        pltpu.VMEM((2, 4, 8, 128), jnp.float32),
        pltpu.SemaphoreType.DMA,
        pltpu.SemaphoreType.DMA,
    ],
    compiler_params=pltpu.CompilerParams(
        use_tc_tiling_on_sc=False, needs_layout_passes=False),
)
def _emb_kernel(idx_hbm, table_hbm, out_hbm, idx_v, rows_v, tbuf, gsem, osem):
    w = lax.axis_index("s") * _NC + lax.axis_index("c")
    pltpu.sync_copy(idx_hbm.at[:, w], idx_v)

    iota = lax.iota(jnp.int32, 16)
    i_el = lax.rem(iota, 8)
    i_eb0 = lax.div(iota, 8)
    i_eb1 = i_eb0 + 2

    def gather(hb, hi, buf):
        pltpu.async_copy(table_hbm.at[idx_v.at[hb, hi]], rows_v.at[buf], gsem)

    def transpose_rows(buf):
        def body(bl, _):
            vbl = jnp.full((16,), bl, jnp.int32)
            v0 = rows_v[buf, bl, pl.ds(0, 16)]
            v1 = rows_v[buf, bl, pl.ds(16, 16)]
            plsc.store_scatter(tbuf.at[buf], [i_eb0, i_el, vbl], v0)
            plsc.store_scatter(tbuf.at[buf], [i_eb1, i_el, vbl], v1)
            return 0
        lax.fori_loop(0, 128, body, 0, unroll=4)

    gather(0, 0, 0)

    def hb_body(hb, _):
        for hi in range(8):
            h = hb * 8 + hi
            cur = hi % 2
            if hi == 7:
                @pl.when(h < HIST - 1)
                def _():
                    gather(hb + 1, 0, 1 - cur)
            else:
                gather(hb, hi + 1, 1 - cur)
            pltpu.make_async_copy(
                table_hbm.at[pl.ds(0, 128)], rows_v.at[cur], gsem).wait()
            @pl.when(h >= 2)
            def _():
                pltpu.make_async_copy(
                    out_hbm.at[0, :, 0], tbuf.at[cur], osem).wait()
            transpose_rows(cur)
            pltpu.async_copy(tbuf.at[cur], out_hbm.at[h, :, w], osem)
        return 0

    lax.fori_loop(0, _HB, hb_body, 0)
    pltpu.make_async_copy(out_hbm.at[0, :, 0], tbuf.at[0], osem).wait()
    pltpu.make_async_copy(out_hbm.at[0, :, 0], tbuf.at[1], osem).wait()


def kernel(indices, table):
    idx4 = indices.T.reshape(_HB, 8, _BB, 128).transpose(0, 2, 1, 3)
    out5 = _emb_kernel(idx4, table)
    return out5.transpose(2, 4, 0, 1, 3).reshape(BATCH, HIST, EMBED_DIM)


# speedup vs baseline: 1.4988x; 1.0005x over previous
import functools

import jax
import jax.numpy as jnp
from jax import lax
from jax.experimental import pallas as pl
from jax.experimental.pallas import tpu as pltpu
from jax.experimental.pallas import tpu_sc as plsc

VOCAB = 1000000
EMBED_DIM = 32
BATCH = 4096
HIST = 200

_INFO = plsc.get_sparse_core_info()
_NC = _INFO.num_cores
_NS = _INFO.num_subcores
_NW = _NC * _NS
_HB = HIST // 8
_BB = BATCH // 128


@functools.partial(
    pl.kernel,
    mesh=plsc.VectorSubcoreMesh(core_axis_name="c", subcore_axis_name="s"),
    out_type=jax.ShapeDtypeStruct((HIST, 4, _BB * 8 * 128), jnp.float32),
    scratch_types=[
        pltpu.VMEM((_HB, 8, 128), jnp.int32),
        pltpu.VMEM((4, 128, EMBED_DIM), jnp.float32),
        pltpu.VMEM((4, 4, 1024), jnp.float32),
        pltpu.SemaphoreType.DMA,
        pltpu.SemaphoreType.DMA,
    ],
    compiler_params=pltpu.CompilerParams(
        use_tc_tiling_on_sc=False, needs_layout_passes=False),
)
def _emb_kernel(idx_hbm, table_hbm, out_hbm, idx_v, rows_v, tbuf, gsem, osem):
    w = lax.axis_index("s") * _NC + lax.axis_index("c")
    pltpu.sync_copy(idx_hbm.at[:, w], idx_v)

    iota = lax.iota(jnp.int32, 16)
    i_q0 = lax.div(iota, 8)
    i_q1 = i_q0 + 2
    pre_r0 = lax.rem(iota, 8) * 128
    pre_r1 = pre_r0

    def gather(hb, hi, slot):
        pltpu.async_copy(table_hbm.at[idx_v.at[hb, hi]], rows_v.at[slot], gsem)

    def transpose_rows(slot):
        def body(bl, _):
            vbl = jnp.full((16,), bl, jnp.int32)
            i_r = pre_r0 + vbl
            v0 = rows_v[slot, bl, pl.ds(0, 16)]
            v1 = rows_v[slot, bl, pl.ds(16, 16)]
            plsc.store_scatter(tbuf.at[slot], [i_q0, i_r], v0)
            plsc.store_scatter(tbuf.at[slot], [i_q1, i_r], v1)
            return 0
        lax.fori_loop(0, 128, body, 0, unroll=4)

    gather(0, 0, 0)
    gather(0, 1, 1)
    gather(0, 2, 2)

    def hb_body(hb, _):
        for hi in range(8):
            h = hb * 8 + hi
            cur = hi % 4
            nhi = (hi + 3) % 8
            nhb = hb + (hi + 3) // 8

            @pl.when(h < HIST - 3)
            def _():
                gather(nhb, nhi, (hi + 3) % 4)

            pltpu.make_async_copy(
                table_hbm.at[pl.ds(0, 128)], rows_v.at[cur], gsem).wait()

            @pl.when(h >= 4)
            def _():
                pltpu.make_async_copy(
                    out_hbm.at[0, :, pl.ds(0, 1024)], tbuf.at[cur], osem).wait()

            transpose_rows(cur)
            pltpu.async_copy(
                tbuf.at[cur], out_hbm.at[h, :, pl.ds(w * 1024, 1024)], osem)
        return 0

    lax.fori_loop(0, _HB, hb_body, 0)
    for slot in range(4):
        pltpu.make_async_copy(
            out_hbm.at[0, :, pl.ds(0, 1024)], tbuf.at[slot], osem).wait()


def kernel(indices, table):
    idx4 = indices.T.reshape(_HB, 8, _BB, 128).transpose(0, 2, 1, 3)
    out3 = _emb_kernel(idx4, table)
    out5 = out3.reshape(HIST, 4, _BB, 8, 128)
    return out5.transpose(2, 4, 0, 1, 3).reshape(BATCH, HIST, EMBED_DIM)


# speedup vs baseline: 1.6313x; 1.0884x over previous
import functools

import jax
import jax.numpy as jnp
from jax import lax
from jax.experimental import pallas as pl
from jax.experimental.pallas import tpu as pltpu
from jax.experimental.pallas import tpu_sc as plsc

VOCAB = 1000000
EMBED_DIM = 32
BATCH = 4096
HIST = 200

_INFO = plsc.get_sparse_core_info()
_NC = _INFO.num_cores
_NS = _INFO.num_subcores
_NW = _NC * _NS
_HB = HIST // 8
_BB = BATCH // 128


@functools.partial(
    pl.kernel,
    mesh=plsc.VectorSubcoreMesh(core_axis_name="c", subcore_axis_name="s"),
    out_type=jax.ShapeDtypeStruct((HIST, 4, _BB * 8 * 128), jnp.float32),
    scratch_types=[
        pltpu.VMEM((_HB, 8, 128), jnp.int32),
        pltpu.VMEM((4, 128, EMBED_DIM), jnp.float32),
        pltpu.VMEM((4, 4, 1024), jnp.float32),
        pltpu.SemaphoreType.DMA,
        pltpu.SemaphoreType.DMA,
    ],
    compiler_params=pltpu.CompilerParams(
        use_tc_tiling_on_sc=False, needs_layout_passes=False),
)
def _emb_kernel(idx_hbm, table_hbm, out_hbm, idx_v, rows_v, tbuf, gsem, osem):
    w = lax.axis_index("s") * _NC + lax.axis_index("c")
    pltpu.sync_copy(idx_hbm.at[:, w], idx_v)

    iota = lax.iota(jnp.int32, 16)
    i_q0 = lax.div(iota, 8)
    i_q1 = i_q0 + 2
    pre_r0 = lax.rem(iota, 8) * 128
    pre_r1 = pre_r0

    def gather(hb, hi, slot):
        pltpu.async_copy(table_hbm.at[idx_v.at[hb, hi]], rows_v.at[slot], gsem)

    def transpose_rows(slot):
        @plsc.parallel_loop(0, 128, 1, unroll=8)
        def _(bl):
            vbl = jnp.full((16,), bl, jnp.int32)
            i_r = pre_r0 + vbl
            v0 = rows_v[slot, bl, pl.ds(0, 16)]
            v1 = rows_v[slot, bl, pl.ds(16, 16)]
            plsc.store_scatter(tbuf.at[slot], [i_q0, i_r], v0)
            plsc.store_scatter(tbuf.at[slot], [i_q1, i_r], v1)

    gather(0, 0, 0)
    gather(0, 1, 1)
    gather(0, 2, 2)

    def hb_body(hb, _):
        for hi in range(8):
            h = hb * 8 + hi
            cur = hi % 4
            nhi = (hi + 3) % 8
            nhb = hb + (hi + 3) // 8

            @pl.when(h < HIST - 3)
            def _():
                gather(nhb, nhi, (hi + 3) % 4)

            pltpu.make_async_copy(
                table_hbm.at[pl.ds(0, 128)], rows_v.at[cur], gsem).wait()

            @pl.when(h >= 4)
            def _():
                pltpu.make_async_copy(
                    out_hbm.at[0, :, pl.ds(0, 1024)], tbuf.at[cur], osem).wait()

            transpose_rows(cur)
            pltpu.async_copy(
                tbuf.at[cur], out_hbm.at[h, :, pl.ds(w * 1024, 1024)], osem)
        return 0

    lax.fori_loop(0, _HB, hb_body, 0)
    for slot in range(4):
        pltpu.make_async_copy(
            out_hbm.at[0, :, pl.ds(0, 1024)], tbuf.at[slot], osem).wait()


def kernel(indices, table):
    idx4 = indices.T.reshape(_HB, 8, _BB, 128).transpose(0, 2, 1, 3)
    out3 = _emb_kernel(idx4, table)
    out5 = out3.reshape(HIST, 4, _BB, 8, 128)
    return out5.transpose(2, 4, 0, 1, 3).reshape(BATCH, HIST, EMBED_DIM)


# speedup vs baseline: 2.5406x; 1.5574x over previous
import functools

import jax
import jax.numpy as jnp
from jax import lax
from jax.experimental import pallas as pl
from jax.experimental.pallas import tpu as pltpu
from jax.experimental.pallas import tpu_sc as plsc

VOCAB = 1000000
EMBED_DIM = 32
BATCH = 4096
HIST = 200

_INFO = plsc.get_sparse_core_info()
_NC = _INFO.num_cores
_NS = _INFO.num_subcores
_NW = _NC * _NS
_HB = HIST // 8
_BB = BATCH // 128


@functools.partial(
    pl.kernel,
    mesh=plsc.VectorSubcoreMesh(core_axis_name="c", subcore_axis_name="s"),
    out_type=jax.ShapeDtypeStruct((HIST, 4, _BB, 8, 128), jnp.float32),
    scratch_types=[
        pltpu.VMEM((_HB, 8, 128), jnp.int32),
        pltpu.VMEM((4, 128, EMBED_DIM), jnp.float32),
        pltpu.VMEM((4, EMBED_DIM, 129), jnp.float32),
        pltpu.SemaphoreType.DMA,
        pltpu.SemaphoreType.DMA,
    ],
    compiler_params=pltpu.CompilerParams(
        use_tc_tiling_on_sc=False, needs_layout_passes=False),
)
def _emb_kernel(idx_hbm, table_hbm, out_hbm, idx_v, rows_v, tbuf, gsem, osem):
    w = lax.axis_index("s") * _NC + lax.axis_index("c")
    pltpu.sync_copy(idx_hbm.at[:, w], idx_v)

    iota = lax.iota(jnp.int32, 16)
    i_e0 = iota
    i_e1 = iota + 16

    def gather(hb, hi, slot):
        pltpu.async_copy(table_hbm.at[idx_v.at[hb, hi]], rows_v.at[slot], gsem)

    def transpose_rows(slot):
        @plsc.parallel_loop(0, 128, 1, unroll=8)
        def _(bl):
            vbl = jnp.full((16,), bl, jnp.int32)
            v0 = rows_v[slot, bl, pl.ds(0, 16)]
            v1 = rows_v[slot, bl, pl.ds(16, 16)]
            plsc.store_scatter(tbuf.at[slot], [i_e0, vbl], v0)
            plsc.store_scatter(tbuf.at[slot], [i_e1, vbl], v1)

    gather(0, 0, 0)
    gather(0, 1, 1)
    gather(0, 2, 2)

    def hb_body(hb, _):
        for hi in range(8):
            h = hb * 8 + hi
            cur = hi % 4
            nhi = (hi + 3) % 8
            nhb = hb + (hi + 3) // 8

            @pl.when(h < HIST - 3)
            def _():
                gather(nhb, nhi, (hi + 3) % 4)

            pltpu.make_async_copy(
                table_hbm.at[pl.ds(0, 128)], rows_v.at[cur], gsem).wait()

            @pl.when(h >= 4)
            def _():
                pltpu.make_async_copy(
                    table_hbm.at[pl.ds(0, 128)], rows_v.at[0], osem).wait()

            transpose_rows(cur)
            for eb in range(4):
                pltpu.async_copy(
                    tbuf.at[cur, pl.ds(8 * eb, 8), pl.ds(0, 128)],
                    out_hbm.at[h, eb, w], osem)
        return 0

    lax.fori_loop(0, _HB, hb_body, 0)
    for _ in range(4):
        pltpu.make_async_copy(
            table_hbm.at[pl.ds(0, 128)], rows_v.at[0], osem).wait()


def kernel(indices, table):
    idx4 = indices.T.reshape(_HB, 8, _BB, 128).transpose(0, 2, 1, 3)
    out5 = _emb_kernel(idx4, table)
    return out5.transpose(2, 4, 0, 1, 3).reshape(BATCH, HIST, EMBED_DIM)


# speedup vs baseline: 4.1754x; 1.6435x over previous
import functools

import jax
import jax.numpy as jnp
from jax import lax
from jax.experimental import pallas as pl
from jax.experimental.pallas import tpu as pltpu
from jax.experimental.pallas import tpu_sc as plsc

VOCAB = 1000000
EMBED_DIM = 32
BATCH = 4096
HIST = 200

_INFO = plsc.get_sparse_core_info()
_NC = _INFO.num_cores
_NS = _INFO.num_subcores
_NW = _NC * _NS
_HB = HIST // 8
_BB = BATCH // 128


_NCHK = VOCAB // 128
_TAIL = VOCAB - _NCHK * 128
_NSLOT = _NCHK // _NW + 2


@functools.partial(
    pl.kernel,
    mesh=plsc.VectorSubcoreMesh(core_axis_name="c", subcore_axis_name="s"),
    out_type=jax.ShapeDtypeStruct((VOCAB * EMBED_DIM,), jnp.float32),
    scratch_types=[
        pltpu.VMEM((EMBED_DIM, 128), jnp.float32),
        pltpu.VMEM((EMBED_DIM, 128), jnp.float32),
        pltpu.VMEM((4096,), jnp.float32),
        pltpu.VMEM((4096,), jnp.float32),
        pltpu.VMEM((_TAIL * EMBED_DIM,), jnp.float32),
        pltpu.SemaphoreType.DMA,
        pltpu.SemaphoreType.DMA,
    ],
    compiler_params=pltpu.CompilerParams(
        use_tc_tiling_on_sc=True, needs_layout_passes=False),
)
def _detile_kernel(tt_hbm, tail_hbm, out_hbm, src_a, src_b, dst_a, dst_b,
                   tail_v, isem, osem):
    w = lax.axis_index("s") * _NC + lax.axis_index("c")
    iota = lax.iota(jnp.int32, 16)

    def start_in(cid, src_x):
        off = pl.multiple_of(cid * 128, 128)
        pltpu.async_copy(tt_hbm.at[:, pl.ds(off, 128)], src_x, isem)

    def drain_in(src_x):
        pltpu.make_async_copy(tt_hbm.at[:, pl.ds(0, 128)], src_x, isem).wait()

    def drain_out(dst_x):
        pltpu.make_async_copy(out_hbm.at[pl.ds(0, 4096)], dst_x, osem).wait()

    def transpose_chunk(src_x, dst_x):
        for eq in range(2):
            i_e = iota + 16 * eq

            def dbody(d, vv):
                vdst = lax.shift_left(vv, 5) + i_e
                val = plsc.load_gather(src_x, [i_e, vv])
                plsc.store_scatter(dst_x, [vdst], val)
                return lax.bitwise_and(vv + 1, 127)

            lax.fori_loop(0, 128, dbody, iota, unroll=8)

    start_in(w, src_a)

    def pair_body(i, _):
        for j in range(2):
            c = i * 2 + j
            src_x = src_a if j == 0 else src_b
            dst_x = dst_a if j == 0 else dst_b
            nsrc_x = src_b if j == 0 else src_a
            cid = c * _NW + w
            ncid = cid + _NW

            @pl.when(ncid < _NCHK)
            def _():
                start_in(ncid, nsrc_x)

            @pl.when(cid < _NCHK)
            def _():
                drain_in(src_x)

                @pl.when(c >= 2)
                def _():
                    drain_out(dst_x)

                transpose_chunk(src_x, dst_x)
                off = pl.multiple_of(cid * 4096, 4096)
                pltpu.async_copy(dst_x, out_hbm.at[pl.ds(off, 4096)], osem)
        return 0

    lax.fori_loop(0, _NSLOT // 2, pair_body, 0)
    drain_out(dst_a)
    drain_out(dst_b)

    @pl.when(w == 0)
    def _():
        pltpu.sync_copy(tail_hbm, tail_v)
        pltpu.sync_copy(
            tail_v, out_hbm.at[pl.ds(_NCHK * 4096, _TAIL * EMBED_DIM)])


@functools.partial(
    pl.kernel,
    mesh=plsc.VectorSubcoreMesh(core_axis_name="c", subcore_axis_name="s"),
    out_type=jax.ShapeDtypeStruct((HIST, 4, _BB, 8, 128), jnp.float32),
    scratch_types=[
        pltpu.VMEM((_HB, 8, 128), jnp.int32),
        pltpu.VMEM((4, 128, EMBED_DIM), jnp.float32),
        pltpu.VMEM((4, EMBED_DIM, 129), jnp.float32),
        pltpu.SemaphoreType.DMA,
        pltpu.SemaphoreType.DMA,
    ],
    compiler_params=pltpu.CompilerParams(
        use_tc_tiling_on_sc=False, needs_layout_passes=False),
)
def _emb_kernel(idx_hbm, table_hbm, out_hbm, idx_v, rows_v, tbuf, gsem, osem):
    w = lax.axis_index("s") * _NC + lax.axis_index("c")
    pltpu.sync_copy(idx_hbm.at[:, w], idx_v)

    iota = lax.iota(jnp.int32, 16)
    i_e0 = iota
    i_e1 = iota + 16

    def gather(hb, hi, slot):
        pltpu.async_copy(table_hbm.at[idx_v.at[hb, hi]], rows_v.at[slot], gsem)

    def transpose_rows(slot):
        @plsc.parallel_loop(0, 128, 1, unroll=8)
        def _(bl):
            vbl = jnp.full((16,), bl, jnp.int32)
            v0 = rows_v[slot, bl, pl.ds(0, 16)]
            v1 = rows_v[slot, bl, pl.ds(16, 16)]
            plsc.store_scatter(tbuf.at[slot], [i_e0, vbl], v0)
            plsc.store_scatter(tbuf.at[slot], [i_e1, vbl], v1)

    gather(0, 0, 0)
    gather(0, 1, 1)
    gather(0, 2, 2)

    def hb_body(hb, _):
        for hi in range(8):
            h = hb * 8 + hi
            cur = hi % 4
            nhi = (hi + 3) % 8
            nhb = hb + (hi + 3) // 8

            @pl.when(h < HIST - 3)
            def _():
                gather(nhb, nhi, (hi + 3) % 4)

            pltpu.make_async_copy(
                table_hbm.at[pl.ds(0, 128)], rows_v.at[cur], gsem).wait()

            @pl.when(h >= 4)
            def _():
                pltpu.make_async_copy(
                    table_hbm.at[pl.ds(0, 128)], rows_v.at[0], osem).wait()

            transpose_rows(cur)
            for eb in range(4):
                pltpu.async_copy(
                    tbuf.at[cur, pl.ds(8 * eb, 8), pl.ds(0, 128)],
                    out_hbm.at[h, eb, w], osem)
        return 0

    lax.fori_loop(0, _HB, hb_body, 0)
    for _ in range(4):
        pltpu.make_async_copy(
            table_hbm.at[pl.ds(0, 128)], rows_v.at[0], osem).wait()


def kernel(indices, table):
    idx4 = indices.T.reshape(_HB, 8, _BB, 128).transpose(0, 2, 1, 3)
    tail = table[_NCHK * 128:, :].reshape(_TAIL * EMBED_DIM)
    tlin = _detile_kernel(table.T, tail)
    out5 = _emb_kernel(idx4, tlin.reshape(VOCAB, EMBED_DIM))
    return out5.transpose(2, 4, 0, 1, 3).reshape(BATCH, HIST, EMBED_DIM)


# speedup vs baseline: 4.2629x; 1.0209x over previous
import functools

import jax
import jax.numpy as jnp
from jax import lax
from jax.experimental import pallas as pl
from jax.experimental.pallas import tpu as pltpu
from jax.experimental.pallas import tpu_sc as plsc

VOCAB = 1000000
EMBED_DIM = 32
BATCH = 4096
HIST = 200

_INFO = plsc.get_sparse_core_info()
_NC = _INFO.num_cores
_NS = _INFO.num_subcores
_NW = _NC * _NS
_HB = HIST // 8
_BB = BATCH // 128


_NCHK = VOCAB // 128
_TAIL = VOCAB - _NCHK * 128
_NSLOT = _NCHK // _NW + 2


@functools.partial(
    pl.kernel,
    mesh=plsc.VectorSubcoreMesh(core_axis_name="c", subcore_axis_name="s"),
    out_type=jax.ShapeDtypeStruct((VOCAB * EMBED_DIM,), jnp.float32),
    scratch_types=[
        pltpu.VMEM((EMBED_DIM, 128), jnp.float32),
        pltpu.VMEM((EMBED_DIM, 128), jnp.float32),
        pltpu.VMEM((4096,), jnp.float32),
        pltpu.VMEM((4096,), jnp.float32),
        pltpu.VMEM((_TAIL * EMBED_DIM,), jnp.float32),
        pltpu.SemaphoreType.DMA,
        pltpu.SemaphoreType.DMA,
    ],
    compiler_params=pltpu.CompilerParams(
        use_tc_tiling_on_sc=True, needs_layout_passes=False),
)
def _detile_kernel(tt_hbm, tail_hbm, out_hbm, src_a, src_b, dst_a, dst_b,
                   tail_v, isem, osem):
    w = lax.axis_index("s") * _NC + lax.axis_index("c")
    iota = lax.iota(jnp.int32, 16)

    def start_in(cid, src_x):
        off = pl.multiple_of(cid * 128, 128)
        pltpu.async_copy(tt_hbm.at[:, pl.ds(off, 128)], src_x, isem)

    def drain_in(src_x):
        pltpu.make_async_copy(tt_hbm.at[:, pl.ds(0, 128)], src_x, isem).wait()

    def drain_out(dst_x):
        pltpu.make_async_copy(out_hbm.at[pl.ds(0, 4096)], dst_x, osem).wait()

    rots = [lax.bitwise_and(iota + d, 15) for d in range(16)]

    def transpose_chunk(src_x, dst_x):
        def vqbody(vq, _):
            vb = jnp.full((16,), 16 * vq, jnp.int32)
            vb32 = jnp.full((16,), 512 * vq, jnp.int32)
            for eq in range(2):
                i_e = iota + 16 * eq
                vbe = vb32 + i_e
                for d in range(16):
                    vv = vb + rots[d]
                    vdst = vbe + lax.shift_left(rots[d], 5)
                    val = plsc.load_gather(src_x, [i_e, vv])
                    plsc.store_scatter(dst_x, [vdst], val)
            return 0

        lax.fori_loop(0, 8, vqbody, 0)

    start_in(w, src_a)

    def pair_body(i, _):
        for j in range(2):
            c = i * 2 + j
            src_x = src_a if j == 0 else src_b
            dst_x = dst_a if j == 0 else dst_b
            nsrc_x = src_b if j == 0 else src_a
            cid = c * _NW + w
            ncid = cid + _NW

            @pl.when(ncid < _NCHK)
            def _():
                start_in(ncid, nsrc_x)

            @pl.when(cid < _NCHK)
            def _():
                drain_in(src_x)

                @pl.when(c >= 2)
                def _():
                    drain_out(dst_x)

                transpose_chunk(src_x, dst_x)
                off = pl.multiple_of(cid * 4096, 4096)
                pltpu.async_copy(dst_x, out_hbm.at[pl.ds(off, 4096)], osem)
        return 0

    lax.fori_loop(0, _NSLOT // 2, pair_body, 0)
    drain_out(dst_a)
    drain_out(dst_b)

    @pl.when(w == 0)
    def _():
        pltpu.sync_copy(tail_hbm, tail_v)
        pltpu.sync_copy(
            tail_v, out_hbm.at[pl.ds(_NCHK * 4096, _TAIL * EMBED_DIM)])


@functools.partial(
    pl.kernel,
    mesh=plsc.VectorSubcoreMesh(core_axis_name="c", subcore_axis_name="s"),
    out_type=jax.ShapeDtypeStruct((HIST, 4, _BB, 8, 128), jnp.float32),
    scratch_types=[
        pltpu.VMEM((_HB, 8, 128), jnp.int32),
        pltpu.VMEM((4, 128, EMBED_DIM), jnp.float32),
        pltpu.VMEM((4, EMBED_DIM, 129), jnp.float32),
        pltpu.SemaphoreType.DMA,
        pltpu.SemaphoreType.DMA,
    ],
    compiler_params=pltpu.CompilerParams(
        use_tc_tiling_on_sc=False, needs_layout_passes=False),
)
def _emb_kernel(idx_hbm, table_hbm, out_hbm, idx_v, rows_v, tbuf, gsem, osem):
    w = lax.axis_index("s") * _NC + lax.axis_index("c")
    pltpu.sync_copy(idx_hbm.at[:, w], idx_v)

    iota = lax.iota(jnp.int32, 16)
    i_e0 = iota
    i_e1 = iota + 16

    def gather(hb, hi, slot):
        pltpu.async_copy(table_hbm.at[idx_v.at[hb, hi]], rows_v.at[slot], gsem)

    def transpose_rows(slot):
        @plsc.parallel_loop(0, 128, 1, unroll=8)
        def _(bl):
            vbl = jnp.full((16,), bl, jnp.int32)
            v0 = rows_v[slot, bl, pl.ds(0, 16)]
            v1 = rows_v[slot, bl, pl.ds(16, 16)]
            plsc.store_scatter(tbuf.at[slot], [i_e0, vbl], v0)
            plsc.store_scatter(tbuf.at[slot], [i_e1, vbl], v1)

    gather(0, 0, 0)
    gather(0, 1, 1)
    gather(0, 2, 2)

    def hb_body(hb, _):
        for hi in range(8):
            h = hb * 8 + hi
            cur = hi % 4
            nhi = (hi + 3) % 8
            nhb = hb + (hi + 3) // 8

            @pl.when(h < HIST - 3)
            def _():
                gather(nhb, nhi, (hi + 3) % 4)

            pltpu.make_async_copy(
                table_hbm.at[pl.ds(0, 128)], rows_v.at[cur], gsem).wait()

            @pl.when(h >= 4)
            def _():
                pltpu.make_async_copy(
                    table_hbm.at[pl.ds(0, 128)], rows_v.at[0], osem).wait()

            transpose_rows(cur)
            for eb in range(4):
                pltpu.async_copy(
                    tbuf.at[cur, pl.ds(8 * eb, 8), pl.ds(0, 128)],
                    out_hbm.at[h, eb, w], osem)
        return 0

    lax.fori_loop(0, _HB, hb_body, 0)
    for _ in range(4):
        pltpu.make_async_copy(
            table_hbm.at[pl.ds(0, 128)], rows_v.at[0], osem).wait()


def kernel(indices, table):
    idx4 = indices.T.reshape(_HB, 8, _BB, 128).transpose(0, 2, 1, 3)
    tail = table[_NCHK * 128:, :].reshape(_TAIL * EMBED_DIM)
    tlin = _detile_kernel(table.T, tail)
    out5 = _emb_kernel(idx4, tlin.reshape(VOCAB, EMBED_DIM))
    return out5.transpose(2, 4, 0, 1, 3).reshape(BATCH, HIST, EMBED_DIM)


# speedup vs baseline: 5.7937x; 1.3591x over previous
import functools

import jax
import jax.numpy as jnp
from jax import lax
from jax.experimental import pallas as pl
from jax.experimental.pallas import tpu as pltpu
from jax.experimental.pallas import tpu_sc as plsc

VOCAB = 1000000
EMBED_DIM = 32
BATCH = 4096
HIST = 200

_INFO = plsc.get_sparse_core_info()
_NC = _INFO.num_cores
_NS = _INFO.num_subcores
_NW = _NC * _NS
_HB = HIST // 8
_BB = BATCH // 128


_CV = 256
_NCHK = VOCAB // _CV
_TAIL = VOCAB - _NCHK * _CV
_NSLOT = _NCHK // _NW + 2
_CW = _CV * EMBED_DIM


@functools.partial(
    pl.kernel,
    mesh=plsc.VectorSubcoreMesh(core_axis_name="c", subcore_axis_name="s"),
    out_type=jax.ShapeDtypeStruct((VOCAB * EMBED_DIM,), jnp.float32),
    scratch_types=[
        pltpu.VMEM((EMBED_DIM, _CV), jnp.float32),
        pltpu.VMEM((EMBED_DIM, _CV), jnp.float32),
        pltpu.VMEM((_CW,), jnp.float32),
        pltpu.VMEM((_CW,), jnp.float32),
        pltpu.VMEM((_TAIL * EMBED_DIM,), jnp.float32),
        pltpu.SemaphoreType.DMA,
        pltpu.SemaphoreType.DMA,
    ],
    compiler_params=pltpu.CompilerParams(
        use_tc_tiling_on_sc=True, needs_layout_passes=False),
)
def _detile_kernel(tt_hbm, tail_hbm, out_hbm, src_a, src_b, dst_a, dst_b,
                   tail_v, isem, osem):
    w = lax.axis_index("s") * _NC + lax.axis_index("c")
    iota = lax.iota(jnp.int32, 16)

    def start_in(cid, src_x):
        off = pl.multiple_of(cid * _CV, _CV)
        pltpu.async_copy(tt_hbm.at[:, pl.ds(off, _CV)], src_x, isem)

    def drain_in(src_x):
        pltpu.make_async_copy(tt_hbm.at[:, pl.ds(0, _CV)], src_x, isem).wait()

    def drain_out(dst_x):
        pltpu.make_async_copy(out_hbm.at[pl.ds(0, _CW)], dst_x, osem).wait()

    rots = [lax.bitwise_and(iota + d, 15) for d in range(16)]
    rot32s = [lax.shift_left(r, 5) for r in rots]

    def transpose_chunk(src_x, dst_x):
        def vqbody(vq, _):
            vb = jnp.full((16,), 16 * vq, jnp.int32)
            vb32 = jnp.full((16,), 512 * vq, jnp.int32)
            for eq in range(2):
                i_e = iota + 16 * eq
                vbe = vb32 + i_e
                for db in range(2):
                    vals = [
                        plsc.load_gather(src_x, [i_e, vb + rots[db * 8 + d8]])
                        for d8 in range(8)
                    ]
                    for d8 in range(8):
                        plsc.store_scatter(
                            dst_x, [vbe + rot32s[db * 8 + d8]], vals[d8])
            return 0

        lax.fori_loop(0, _CV // 16, vqbody, 0)

    start_in(w, src_a)

    def pair_body(i, _):
        for j in range(2):
            c = i * 2 + j
            src_x = src_a if j == 0 else src_b
            dst_x = dst_a if j == 0 else dst_b
            nsrc_x = src_b if j == 0 else src_a
            cid = c * _NW + w
            ncid = cid + _NW

            @pl.when(ncid < _NCHK)
            def _():
                start_in(ncid, nsrc_x)

            @pl.when(cid < _NCHK)
            def _():
                drain_in(src_x)

                @pl.when(c >= 2)
                def _():
                    drain_out(dst_x)

                transpose_chunk(src_x, dst_x)
                off = pl.multiple_of(cid * _CW, _CW)
                pltpu.async_copy(dst_x, out_hbm.at[pl.ds(off, _CW)], osem)
        return 0

    lax.fori_loop(0, _NSLOT // 2, pair_body, 0)
    drain_out(dst_a)
    drain_out(dst_b)

    @pl.when(w == 0)
    def _():
        pltpu.sync_copy(tail_hbm, tail_v)
        pltpu.sync_copy(
            tail_v, out_hbm.at[pl.ds(_NCHK * _CW, _TAIL * EMBED_DIM)])


@functools.partial(
    pl.kernel,
    mesh=plsc.VectorSubcoreMesh(core_axis_name="c", subcore_axis_name="s"),
    out_type=jax.ShapeDtypeStruct((HIST, 4, _BB, 8, 128), jnp.float32),
    scratch_types=[
        pltpu.VMEM((_HB, 8, 128), jnp.int32),
        pltpu.VMEM((4, 128, EMBED_DIM), jnp.float32),
        pltpu.VMEM((4, EMBED_DIM, 129), jnp.float32),
        pltpu.SemaphoreType.DMA,
        pltpu.SemaphoreType.DMA,
    ],
    compiler_params=pltpu.CompilerParams(
        use_tc_tiling_on_sc=False, needs_layout_passes=False),
)
def _emb_kernel(idx_hbm, table_hbm, out_hbm, idx_v, rows_v, tbuf, gsem, osem):
    w = lax.axis_index("s") * _NC + lax.axis_index("c")
    pltpu.sync_copy(idx_hbm.at[:, w], idx_v)

    iota = lax.iota(jnp.int32, 16)
    i_e0 = iota
    i_e1 = iota + 16

    def gather(hb, hi, slot):
        pltpu.async_copy(table_hbm.at[idx_v.at[hb, hi]], rows_v.at[slot], gsem)

    def transpose_rows(slot):
        @plsc.parallel_loop(0, 128, 1, unroll=8)
        def _(bl):
            vbl = jnp.full((16,), bl, jnp.int32)
            v0 = rows_v[slot, bl, pl.ds(0, 16)]
            v1 = rows_v[slot, bl, pl.ds(16, 16)]
            plsc.store_scatter(tbuf.at[slot], [i_e0, vbl], v0)
            plsc.store_scatter(tbuf.at[slot], [i_e1, vbl], v1)

    gather(0, 0, 0)
    gather(0, 1, 1)
    gather(0, 2, 2)

    def hb_body(hb, _):
        for hi in range(8):
            h = hb * 8 + hi
            cur = hi % 4
            nhi = (hi + 3) % 8
            nhb = hb + (hi + 3) // 8

            @pl.when(h < HIST - 3)
            def _():
                gather(nhb, nhi, (hi + 3) % 4)

            pltpu.make_async_copy(
                table_hbm.at[pl.ds(0, 128)], rows_v.at[cur], gsem).wait()

            @pl.when(h >= 4)
            def _():
                pltpu.make_async_copy(
                    table_hbm.at[pl.ds(0, 128)], rows_v.at[0], osem).wait()

            transpose_rows(cur)
            for eb in range(4):
                pltpu.async_copy(
                    tbuf.at[cur, pl.ds(8 * eb, 8), pl.ds(0, 128)],
                    out_hbm.at[h, eb, w], osem)
        return 0

    lax.fori_loop(0, _HB, hb_body, 0)
    for _ in range(4):
        pltpu.make_async_copy(
            table_hbm.at[pl.ds(0, 128)], rows_v.at[0], osem).wait()


def kernel(indices, table):
    idx4 = indices.T.reshape(_HB, 8, _BB, 128).transpose(0, 2, 1, 3)
    tail = table[_NCHK * _CV:, :].reshape(_TAIL * EMBED_DIM)
    tlin = _detile_kernel(table.T, tail)
    out5 = _emb_kernel(idx4, tlin.reshape(VOCAB, EMBED_DIM))
    return out5.transpose(2, 4, 0, 1, 3).reshape(BATCH, HIST, EMBED_DIM)


# speedup vs baseline: 5.8162x; 1.0039x over previous
import functools

import jax
import jax.numpy as jnp
from jax import lax
from jax.experimental import pallas as pl
from jax.experimental.pallas import tpu as pltpu
from jax.experimental.pallas import tpu_sc as plsc

VOCAB = 1000000
EMBED_DIM = 32
BATCH = 4096
HIST = 200

_INFO = plsc.get_sparse_core_info()
_NC = _INFO.num_cores
_NS = _INFO.num_subcores
_NW = _NC * _NS
_HB = HIST // 8
_BB = BATCH // 128


_CV = 512
_NCHK = VOCAB // _CV
_TAIL = VOCAB - _NCHK * _CV
_NSLOT = _NCHK // _NW + 2
_CW = _CV * EMBED_DIM


@functools.partial(
    pl.kernel,
    mesh=plsc.VectorSubcoreMesh(core_axis_name="c", subcore_axis_name="s"),
    out_type=jax.ShapeDtypeStruct((VOCAB * EMBED_DIM,), jnp.float32),
    scratch_types=[
        pltpu.VMEM((EMBED_DIM, _CV), jnp.float32),
        pltpu.VMEM((EMBED_DIM, _CV), jnp.float32),
        pltpu.VMEM((_CW,), jnp.float32),
        pltpu.VMEM((_CW,), jnp.float32),
        pltpu.VMEM((_TAIL * EMBED_DIM,), jnp.float32),
        pltpu.SemaphoreType.DMA,
        pltpu.SemaphoreType.DMA,
    ],
    compiler_params=pltpu.CompilerParams(
        use_tc_tiling_on_sc=True, needs_layout_passes=False),
)
def _detile_kernel(tt_hbm, tail_hbm, out_hbm, src_a, src_b, dst_a, dst_b,
                   tail_v, isem, osem):
    w = lax.axis_index("s") * _NC + lax.axis_index("c")
    iota = lax.iota(jnp.int32, 16)

    def start_in(cid, src_x):
        off = pl.multiple_of(cid * _CV, _CV)
        pltpu.async_copy(tt_hbm.at[:, pl.ds(off, _CV)], src_x, isem)

    def drain_in(src_x):
        pltpu.make_async_copy(tt_hbm.at[:, pl.ds(0, _CV)], src_x, isem).wait()

    def drain_out(dst_x):
        pltpu.make_async_copy(out_hbm.at[pl.ds(0, _CW)], dst_x, osem).wait()

    rots = [lax.bitwise_and(iota + d, 15) for d in range(16)]
    rot32s = [lax.shift_left(r, 5) for r in rots]

    def transpose_chunk(src_x, dst_x):
        def vqbody(vq, _):
            vb = jnp.full((16,), 16 * vq, jnp.int32)
            vb32 = jnp.full((16,), 512 * vq, jnp.int32)
            for eq in range(2):
                i_e = iota + 16 * eq
                vbe = vb32 + i_e
                for db in range(2):
                    vals = [
                        plsc.load_gather(src_x, [i_e, vb + rots[db * 8 + d8]])
                        for d8 in range(8)
                    ]
                    for d8 in range(8):
                        plsc.store_scatter(
                            dst_x, [vbe + rot32s[db * 8 + d8]], vals[d8])
            return 0

        lax.fori_loop(0, _CV // 16, vqbody, 0)

    start_in(w, src_a)

    def pair_body(i, _):
        for j in range(2):
            c = i * 2 + j
            src_x = src_a if j == 0 else src_b
            dst_x = dst_a if j == 0 else dst_b
            nsrc_x = src_b if j == 0 else src_a
            cid = c * _NW + w
            ncid = cid + _NW

            @pl.when(ncid < _NCHK)
            def _():
                start_in(ncid, nsrc_x)

            @pl.when(cid < _NCHK)
            def _():
                drain_in(src_x)

                @pl.when(c >= 2)
                def _():
                    drain_out(dst_x)

                transpose_chunk(src_x, dst_x)
                off = pl.multiple_of(cid * _CW, _CW)
                pltpu.async_copy(dst_x, out_hbm.at[pl.ds(off, _CW)], osem)
        return 0

    lax.fori_loop(0, _NSLOT // 2, pair_body, 0)
    drain_out(dst_a)
    drain_out(dst_b)

    @pl.when(w == 0)
    def _():
        pltpu.sync_copy(tail_hbm, tail_v)
        pltpu.sync_copy(
            tail_v, out_hbm.at[pl.ds(_NCHK * _CW, _TAIL * EMBED_DIM)])


@functools.partial(
    pl.kernel,
    mesh=plsc.VectorSubcoreMesh(core_axis_name="c", subcore_axis_name="s"),
    out_type=jax.ShapeDtypeStruct((HIST, 4, _BB, 8, 128), jnp.float32),
    scratch_types=[
        pltpu.VMEM((_HB, 8, 128), jnp.int32),
        pltpu.VMEM((4, 128, EMBED_DIM), jnp.float32),
        pltpu.VMEM((4, EMBED_DIM, 129), jnp.float32),
        pltpu.SemaphoreType.DMA,
        pltpu.SemaphoreType.DMA,
    ],
    compiler_params=pltpu.CompilerParams(
        use_tc_tiling_on_sc=False, needs_layout_passes=False),
)
def _emb_kernel(idx_hbm, table_hbm, out_hbm, idx_v, rows_v, tbuf, gsem, osem):
    w = lax.axis_index("s") * _NC + lax.axis_index("c")
    pltpu.sync_copy(idx_hbm.at[:, w], idx_v)

    iota = lax.iota(jnp.int32, 16)
    i_e0 = iota
    i_e1 = iota + 16

    def gather(hb, hi, slot):
        pltpu.async_copy(table_hbm.at[idx_v.at[hb, hi]], rows_v.at[slot], gsem)

    def transpose_rows(slot):
        @plsc.parallel_loop(0, 128, 1, unroll=8)
        def _(bl):
            vbl = jnp.full((16,), bl, jnp.int32)
            v0 = rows_v[slot, bl, pl.ds(0, 16)]
            v1 = rows_v[slot, bl, pl.ds(16, 16)]
            plsc.store_scatter(tbuf.at[slot], [i_e0, vbl], v0)
            plsc.store_scatter(tbuf.at[slot], [i_e1, vbl], v1)

    gather(0, 0, 0)
    gather(0, 1, 1)
    gather(0, 2, 2)

    def hb_body(hb, _):
        for hi in range(8):
            h = hb * 8 + hi
            cur = hi % 4
            nhi = (hi + 3) % 8
            nhb = hb + (hi + 3) // 8

            @pl.when(h < HIST - 3)
            def _():
                gather(nhb, nhi, (hi + 3) % 4)

            pltpu.make_async_copy(
                table_hbm.at[pl.ds(0, 128)], rows_v.at[cur], gsem).wait()

            @pl.when(h >= 4)
            def _():
                pltpu.make_async_copy(
                    table_hbm.at[pl.ds(0, 128)], rows_v.at[0], osem).wait()

            transpose_rows(cur)
            for eb in range(4):
                pltpu.async_copy(
                    tbuf.at[cur, pl.ds(8 * eb, 8), pl.ds(0, 128)],
                    out_hbm.at[h, eb, w], osem)
        return 0

    lax.fori_loop(0, _HB, hb_body, 0)
    for _ in range(4):
        pltpu.make_async_copy(
            table_hbm.at[pl.ds(0, 128)], rows_v.at[0], osem).wait()


def kernel(indices, table):
    idx4 = indices.T.reshape(_HB, 8, _BB, 128).transpose(0, 2, 1, 3)
    tail = table[_NCHK * _CV:, :].reshape(_TAIL * EMBED_DIM)
    tlin = _detile_kernel(table.T, tail)
    out5 = _emb_kernel(idx4, tlin.reshape(VOCAB, EMBED_DIM))
    return out5.transpose(2, 4, 0, 1, 3).reshape(BATCH, HIST, EMBED_DIM)
